# Initial kernel scaffold; baseline (speedup 1.0000x reference)
import functools

import jax
import jax.numpy as jnp
import numpy as np
from jax import lax
from jax.experimental import pallas as pl
from jax.experimental.pallas import tpu as pltpu
from jax.experimental.pallas import tpu_sc as plsc

_K = 100
_NC = _K + 1
_NI = 100000
_D = 32
_B = 1024
_L = 50
_G = 16
_NG = _B // _G
_GW = 2
_CCH = 13
_SCH = 7

_CONSTS = {}


def _get_consts():
    if not _CONSTS:
        g = jax.random.gumbel(jax.random.key(42), (_B, _NI), dtype=jnp.float32)
        _, top_idx = jax.lax.top_k(g, _NC)
        top_idx = np.asarray(top_idx).astype(np.int32)
        noise = jax.random.uniform(jax.random.key(7), (_B, _NC), dtype=jnp.float32)
        noise_sm = np.asarray(jax.nn.softmax(noise, axis=-1))
        top_g = np.ascontiguousarray(
            top_idx.reshape(_NG, _G, _NC).transpose(0, 2, 1))
        noise_g = np.ascontiguousarray(
            noise_sm.reshape(_NG, _G, _NC).transpose(0, 2, 1))
        _CONSTS["top"] = jnp.asarray(top_g)
        _CONSTS["noise"] = jnp.asarray(noise_g.reshape(_NG * _NC, _G))
    return _CONSTS["top"], _CONSTS["noise"]


def _sc_body(seq_hbm, tgt_hbm, top_hbm, emb_hbm, cand_out, seq_out,
             top_v, tgt_v, cidx_v, crows_v, sidx_v, srows_v, sem):
    wid = lax.axis_index("s") * 2 + lax.axis_index("c")
    zeros16 = jnp.zeros((_G,), jnp.int32)
    for gi in range(_GW):
        grp = wid * _GW + gi
        base = grp * _G
        pltpu.sync_copy(tgt_hbm.at[pl.ds(base, _G)], tgt_v)
        pltpu.sync_copy(top_hbm.at[grp], top_v)
        for z in range(_SCH * 128 - _G * _L, _SCH * 128, _G):
            sidx_v[pl.ds(z, _G)] = zeros16
        pltpu.sync_copy(seq_hbm.at[pl.ds(base * _L, _G * _L)],
                        sidx_v.at[pl.ds(0, _G * _L)])
        tgt = tgt_v[...]
        cidx_v[pl.ds(0, _G)] = tgt
        col_prev = top_v[0]
        cum = jnp.where(col_prev == tgt, 1, 0)
        for j in range(1, _NC):
            col = top_v[j]
            cidx_v[pl.ds(j * _G, _G)] = jnp.where(cum > 0, col, col_prev)
            cum = cum + jnp.where(col == tgt, 1, 0)
            col_prev = col
        for j in range(_NC, _CCH * 128 // _G):
            cidx_v[pl.ds(j * _G, _G)] = zeros16
        copies = []
        for c in range(_CCH):
            copies.append(pltpu.make_async_copy(
                emb_hbm.at[cidx_v.at[pl.ds(c * 128, 128)]],
                crows_v.at[pl.ds(c * 128, 128)], sem))
        for c in range(_SCH):
            copies.append(pltpu.make_async_copy(
                emb_hbm.at[sidx_v.at[pl.ds(c * 128, 128)]],
                srows_v.at[pl.ds(c * 128, 128)], sem))
        for cp in copies:
            cp.start()
        for cp in copies:
            cp.wait()
        pltpu.sync_copy(crows_v.at[pl.ds(0, _G * _NC)],
                        cand_out.at[pl.ds(grp * _G * _NC, _G * _NC)])
        pltpu.sync_copy(srows_v.at[pl.ds(0, _G * _L)],
                        seq_out.at[pl.ds(grp * _G * _L, _G * _L)])


def _sc_gather(seq_flat, tgt, top_g, item_emb):
    mesh = plsc.VectorSubcoreMesh(core_axis_name="c", subcore_axis_name="s")
    out_type = (
        jax.ShapeDtypeStruct((_NG * _NC * _G, _D), jnp.float32),
        jax.ShapeDtypeStruct((_B * _L, _D), jnp.float32),
    )
    run = functools.partial(
        pl.kernel, mesh=mesh, out_type=out_type,
        scratch_types=[
            pltpu.VMEM((_NC, _G), jnp.int32),
            pltpu.VMEM((_G,), jnp.int32),
            pltpu.VMEM((_CCH * 128,), jnp.int32),
            pltpu.VMEM((_CCH * 128, _D), jnp.float32),
            pltpu.VMEM((_SCH * 128,), jnp.int32),
            pltpu.VMEM((_SCH * 128, _D), jnp.float32),
            pltpu.SemaphoreType.DMA,
        ],
    )(_sc_body)
    return run(seq_flat, tgt, top_g, item_emb)


def _tc_body(seq_ref, cand_ref, ts_ref, tis_ref, tt_ref, len_ref, noise_ref,
             out_ref, acc_ref):
    gidx = pl.program_id(0)
    ts = ts_ref[...]
    tis = tis_ref[...]
    tt = tt_ref[...]
    eff = jnp.maximum(len_ref[...], 1)
    pos = lax.broadcasted_iota(jnp.int32, (_G, _L), 1)
    mask = (pos < eff).astype(jnp.float32)
    decay = jnp.exp(-jnp.abs(tt - ts) * 0.1)
    w = decay * (1.0 + 0.01 * tis) * mask
    wsum = jnp.sum(w, axis=1, keepdims=True)
    seq = seq_ref[...]
    user = jnp.sum(seq * w[:, :, None], axis=1) / (wsum + 1e-6)
    cand = cand_ref[...]
    logits = jnp.sum(cand * user[None, :, :], axis=2)
    m = jnp.max(logits, axis=0, keepdims=True)
    e = jnp.exp(logits - m)
    actual = e / jnp.sum(e, axis=0, keepdims=True)
    noise = noise_ref[...]
    deno = float(_K) * noise + actual + 1e-6
    kidx = lax.broadcasted_iota(jnp.int32, (_NC, _G), 0)
    likeli = jnp.where(kidx == 0, actual, noise) / deno
    part = jnp.sum(jnp.log(likeli))

    @pl.when(gidx == 0)
    def _init():
        acc_ref[0] = 0.0

    acc_ref[0] += part

    @pl.when(gidx == _NG - 1)
    def _fin():
        out_ref[0, 0] = -acc_ref[0] / float(_B * _NC)


def _tc_score(seq3, cand3, ts, tis, tt2, len2, noise2):
    return pl.pallas_call(
        _tc_body,
        grid=(_NG,),
        in_specs=[
            pl.BlockSpec((_G, _L, _D), lambda g: (g, 0, 0)),
            pl.BlockSpec((_NC, _G, _D), lambda g: (g, 0, 0)),
            pl.BlockSpec((_G, _L), lambda g: (g, 0)),
            pl.BlockSpec((_G, _L), lambda g: (g, 0)),
            pl.BlockSpec((_G, 1), lambda g: (g, 0)),
            pl.BlockSpec((_G, 1), lambda g: (g, 0)),
            pl.BlockSpec((_NC, _G), lambda g: (g, 0)),
        ],
        out_specs=pl.BlockSpec((1, 1), lambda g: (0, 0)),
        out_shape=jax.ShapeDtypeStruct((1, 1), jnp.float32),
        scratch_shapes=[pltpu.SMEM((1,), jnp.float32)],
    )(seq3, cand3, ts, tis, tt2, len2, noise2)


def kernel(item_seq, item_seq_len, target_id, time_seq, time_interval_seq,
           target_time, item_emb):
    top_g, noise2 = _get_consts()
    seq_flat = item_seq.astype(jnp.int32).reshape(_B * _L)
    tgt = target_id.astype(jnp.int32)
    cand_emb, seq_emb = _sc_gather(seq_flat, tgt, top_g, item_emb)
    seq3 = seq_emb.reshape(_B, _L, _D)
    cand3 = cand_emb.reshape(_NG * _NC, _G, _D)
    tt2 = target_time.reshape(_B, 1)
    len2 = item_seq_len.astype(jnp.int32).reshape(_B, 1)
    loss = _tc_score(seq3, cand3, time_seq, time_interval_seq, tt2, len2,
                     noise2)
    return loss[0, 0]


# speedup vs baseline: 61.3028x; 61.3028x over previous
import functools

import jax
import jax.numpy as jnp
import numpy as np
from jax import lax
from jax.experimental import pallas as pl
from jax.experimental.pallas import tpu as pltpu
from jax.experimental.pallas import tpu_sc as plsc

_K = 100
_NC = _K + 1
_NI = 100000
_D = 32
_B = 1024
_L = 50
_G = 16
_NG = _B // _G
_GW = 2
_CCH = 13
_SCH = 7

def _threefry_bits_np(start, stop, k0, k1):
    ks0 = np.uint32(k0)
    ks1 = np.uint32(k1)
    ks2 = np.uint32(ks0 ^ ks1 ^ np.uint32(0x1BD11BDA))
    ks = (ks0, ks1, ks2)
    rot = ((13, 15, 26, 6), (17, 29, 16, 24))
    x1 = np.arange(start, stop, dtype=np.uint32)
    x0 = np.full(stop - start, ks0, np.uint32)
    x1 += ks1
    for i in range(5):
        for r in rot[i % 2]:
            x0 += x1
            x1 = (x1 << np.uint32(r)) | (x1 >> np.uint32(32 - r))
            x1 ^= x0
        x0 += ks[(i + 1) % 3]
        x1 += ks[(i + 2) % 3]
        x1 += np.uint32(i + 1)
    return x0 ^ x1


def _bits_to_unit_float_np(bits):
    fb = (bits >> np.uint32(9)) | np.uint32(0x3F800000)
    return fb.view(np.float32) - np.float32(1.0)


def _build_consts():
    top_idx = np.empty((_B, _NC), np.int32)
    tiny = np.float32(np.finfo(np.float32).tiny)
    span = np.float32(1.0) - tiny
    rows_per = 64
    for r0 in range(0, _B, rows_per):
        bits = _threefry_bits_np(r0 * _NI, (r0 + rows_per) * _NI, 0, 42)
        u = _bits_to_unit_float_np(bits) * span + tiny
        u = np.maximum(tiny, u)
        g = -np.log(-np.log(u)).reshape(rows_per, _NI)
        part = np.argpartition(-g, _NC - 1, axis=1)[:, :_NC]
        vals = np.take_along_axis(g, part, axis=1)
        for i in range(rows_per):
            order = np.lexsort((part[i], -vals[i]))
            top_idx[r0 + i] = part[i][order]
    nbits = _threefry_bits_np(0, _B * _NC, 0, 7)
    noise = _bits_to_unit_float_np(nbits).reshape(_B, _NC)
    e = np.exp(noise - noise.max(axis=1, keepdims=True))
    noise_sm = (e / e.sum(axis=1, keepdims=True)).astype(np.float32)
    top_g = np.ascontiguousarray(
        top_idx.reshape(_NG, _G, _NC).transpose(0, 2, 1))
    noise_g = np.ascontiguousarray(
        noise_sm.reshape(_NG, _G, _NC).transpose(0, 2, 1))
    return top_g, noise_g


_TOP_G_NP, _NOISE2_NP = _build_consts()


def _sc_body(seq_hbm, tgt_hbm, top_hbm, emb_hbm, cand_out, seq_out,
             top_v, tgt_v, cidx_v, crows_v, sidx_v, srows_v, sem):
    wid = lax.axis_index("s") * 2 + lax.axis_index("c")
    zeros16 = jnp.zeros((_G,), jnp.int32)
    for gi in range(_GW):
        grp = wid * _GW + gi
        base = grp * _G
        pltpu.sync_copy(tgt_hbm.at[pl.ds(base, _G)], tgt_v)
        pltpu.sync_copy(top_hbm.at[grp], top_v)
        for z in range(_SCH * 128 - _G * _L, _SCH * 128, _G):
            sidx_v[pl.ds(z, _G)] = zeros16
        pltpu.sync_copy(seq_hbm.at[pl.ds(base * _L, _G * _L)],
                        sidx_v.at[pl.ds(0, _G * _L)])
        tgt = tgt_v[...]
        cidx_v[pl.ds(0, _G)] = tgt
        col_prev = top_v[0]
        cum = jnp.where(col_prev == tgt, 1, 0)
        for j in range(1, _NC):
            col = top_v[j]
            cidx_v[pl.ds(j * _G, _G)] = jnp.where(cum > 0, col, col_prev)
            cum = cum + jnp.where(col == tgt, 1, 0)
            col_prev = col
        for j in range(_NC, _CCH * 128 // _G):
            cidx_v[pl.ds(j * _G, _G)] = zeros16
        copies = []
        for c in range(_CCH):
            copies.append(pltpu.make_async_copy(
                emb_hbm.at[cidx_v.at[pl.ds(c * 128, 128)]],
                crows_v.at[pl.ds(c * 128, 128)], sem))
        for c in range(_SCH):
            copies.append(pltpu.make_async_copy(
                emb_hbm.at[sidx_v.at[pl.ds(c * 128, 128)]],
                srows_v.at[pl.ds(c * 128, 128)], sem))
        for cp in copies:
            cp.start()
        for cp in copies:
            cp.wait()
        pltpu.sync_copy(crows_v.at[pl.ds(0, _G * _NC)],
                        cand_out.at[pl.ds(grp * _G * _NC, _G * _NC)])
        pltpu.sync_copy(srows_v.at[pl.ds(0, _G * _L)],
                        seq_out.at[pl.ds(grp * _G * _L, _G * _L)])


def _sc_gather(seq_flat, tgt, top_g, item_emb):
    mesh = plsc.VectorSubcoreMesh(core_axis_name="c", subcore_axis_name="s")
    out_type = (
        jax.ShapeDtypeStruct((_NG * _NC * _G, _D), jnp.float32),
        jax.ShapeDtypeStruct((_B * _L, _D), jnp.float32),
    )
    run = functools.partial(
        pl.kernel, mesh=mesh, out_type=out_type,
        compiler_params=pltpu.CompilerParams(use_tc_tiling_on_sc=False),
        scratch_types=[
            pltpu.VMEM((_NC, _G), jnp.int32),
            pltpu.VMEM((_G,), jnp.int32),
            pltpu.VMEM((_CCH * 128,), jnp.int32),
            pltpu.VMEM((_CCH * 128, _D), jnp.float32),
            pltpu.VMEM((_SCH * 128,), jnp.int32),
            pltpu.VMEM((_SCH * 128, _D), jnp.float32),
            pltpu.SemaphoreType.DMA,
        ],
    )(_sc_body)
    return run(seq_flat, tgt, top_g, item_emb)


def _tc_body(seq_ref, cand_ref, ts_ref, tis_ref, tt_ref, len_ref, noise_ref,
             out_ref, acc_ref):
    gidx = pl.program_id(0)
    ts = ts_ref[...]
    tis = tis_ref[...]
    tt = tt_ref[...]
    eff = jnp.maximum(len_ref[...], 1)
    pos = lax.broadcasted_iota(jnp.int32, (_G, _L), 1)
    mask = (pos < eff).astype(jnp.float32)
    decay = jnp.exp(-jnp.abs(tt - ts) * 0.1)
    w = decay * (1.0 + 0.01 * tis) * mask
    wsum = jnp.sum(w, axis=1, keepdims=True)
    seq = seq_ref[...]
    user = jnp.sum(seq * w[:, :, None], axis=1) / (wsum + 1e-6)
    cand = cand_ref[...]
    logits = jnp.sum(cand * user[None, :, :], axis=2)
    m = jnp.max(logits, axis=0, keepdims=True)
    e = jnp.exp(logits - m)
    actual = e / jnp.sum(e, axis=0, keepdims=True)
    noise = noise_ref[0]
    deno = float(_K) * noise + actual + 1e-6
    kidx = lax.broadcasted_iota(jnp.int32, (_NC, _G), 0)
    likeli = jnp.where(kidx == 0, actual, noise) / deno
    part = jnp.sum(jnp.log(likeli))

    @pl.when(gidx == 0)
    def _init():
        acc_ref[0] = 0.0

    acc_ref[0] += part

    @pl.when(gidx == _NG - 1)
    def _fin():
        out_ref[...] = jnp.full((1, 1), -acc_ref[0] / float(_B * _NC),
                                jnp.float32)


def _tc_score(seq3, cand3, ts, tis, tt2, len2, noise2):
    return pl.pallas_call(
        _tc_body,
        grid=(_NG,),
        in_specs=[
            pl.BlockSpec((_G, _L, _D), lambda g: (g, 0, 0)),
            pl.BlockSpec((_NC, _G, _D), lambda g: (g, 0, 0)),
            pl.BlockSpec((_G, _L), lambda g: (g, 0)),
            pl.BlockSpec((_G, _L), lambda g: (g, 0)),
            pl.BlockSpec((_G, 1), lambda g: (g, 0)),
            pl.BlockSpec((_G, 1), lambda g: (g, 0)),
            pl.BlockSpec((1, _NC, _G), lambda g: (g, 0, 0)),
        ],
        out_specs=pl.BlockSpec((1, 1), lambda g: (0, 0)),
        out_shape=jax.ShapeDtypeStruct((1, 1), jnp.float32),
        scratch_shapes=[pltpu.SMEM((1,), jnp.float32)],
    )(seq3, cand3, ts, tis, tt2, len2, noise2)


def kernel(item_seq, item_seq_len, target_id, time_seq, time_interval_seq,
           target_time, item_emb):
    top_g = jnp.asarray(_TOP_G_NP)
    noise2 = jnp.asarray(_NOISE2_NP)
    seq_flat = item_seq.astype(jnp.int32).reshape(_B * _L)
    tgt = target_id.astype(jnp.int32)
    cand_emb, seq_emb = _sc_gather(seq_flat, tgt, top_g, item_emb)
    seq3 = seq_emb.reshape(_B, _L, _D)
    cand3 = cand_emb.reshape(_NG * _NC, _G, _D)
    tt2 = target_time.reshape(_B, 1)
    len2 = item_seq_len.astype(jnp.int32).reshape(_B, 1)
    loss = _tc_score(seq3, cand3, time_seq, time_interval_seq, tt2, len2,
                     noise2)
    return loss[0, 0]


# speedup vs baseline: 91.3664x; 1.4904x over previous
import functools

import jax
import jax.numpy as jnp
import numpy as np
from jax import lax
from jax.experimental import pallas as pl
from jax.experimental.pallas import tpu as pltpu
from jax.experimental.pallas import tpu_sc as plsc

_K = 100
_NC = _K + 1
_NI = 100000
_D = 32
_B = 1024
_L = 50
_G = 16
_NG = _B // _G
_GW = 2
_CCH = 13
_SCH = 7


def _threefry_bits_np(start, stop, k0, k1):
    ks0 = np.uint32(k0)
    ks1 = np.uint32(k1)
    ks2 = np.uint32(ks0 ^ ks1 ^ np.uint32(0x1BD11BDA))
    ks = (ks0, ks1, ks2)
    rot = ((13, 15, 26, 6), (17, 29, 16, 24))
    x1 = np.arange(start, stop, dtype=np.uint32)
    x0 = np.full(stop - start, ks0, np.uint32)
    x1 += ks1
    for i in range(5):
        for r in rot[i % 2]:
            x0 += x1
            x1 = (x1 << np.uint32(r)) | (x1 >> np.uint32(32 - r))
            x1 ^= x0
        x0 += ks[(i + 1) % 3]
        x1 += ks[(i + 2) % 3]
        x1 += np.uint32(i + 1)
    return x0 ^ x1


def _bits_to_unit_float_np(bits):
    fb = (bits >> np.uint32(9)) | np.uint32(0x3F800000)
    return fb.view(np.float32) - np.float32(1.0)


def _build_consts():
    top_idx = np.empty((_B, _NC), np.int32)
    tiny = np.float32(np.finfo(np.float32).tiny)
    span = np.float32(1.0) - tiny
    rows_per = 64
    for r0 in range(0, _B, rows_per):
        bits = _threefry_bits_np(r0 * _NI, (r0 + rows_per) * _NI, 0, 42)
        u = _bits_to_unit_float_np(bits) * span + tiny
        u = np.maximum(tiny, u)
        g = -np.log(-np.log(u)).reshape(rows_per, _NI)
        part = np.argpartition(-g, _NC - 1, axis=1)[:, :_NC]
        vals = np.take_along_axis(g, part, axis=1)
        for i in range(rows_per):
            order = np.lexsort((part[i], -vals[i]))
            top_idx[r0 + i] = part[i][order]
    nbits = _threefry_bits_np(0, _B * _NC, 0, 7)
    noise = _bits_to_unit_float_np(nbits).reshape(_B, _NC)
    e = np.exp(noise - noise.max(axis=1, keepdims=True))
    noise_sm = (e / e.sum(axis=1, keepdims=True)).astype(np.float32)
    top_g = np.ascontiguousarray(
        top_idx.reshape(_NG, _G, _NC).transpose(0, 2, 1))
    noise_g = np.ascontiguousarray(
        noise_sm.reshape(_NG, _G, _NC).transpose(0, 2, 1))
    return top_g, noise_g


_TOP_G_NP, _NOISE_G_NP = _build_consts()


def _sc_body(seq_hbm, tgt_hbm, top_hbm, emb_hbm, cand_out, seq_out,
             top_v, tgt_v, cidx_v, crows_v, sidx_v, srows_v, sem):
    wid = lax.axis_index("s") * 2 + lax.axis_index("c")
    for gi in range(_GW):
        grp = wid * _GW + gi
        base = grp * _G
        stage = [
            pltpu.make_async_copy(tgt_hbm.at[pl.ds(base, _G)], tgt_v, sem),
            pltpu.make_async_copy(top_hbm.at[grp], top_v, sem),
            pltpu.make_async_copy(
                seq_hbm.at[pl.ds(base * _L, _G * _L)], sidx_v, sem),
        ]
        for cp in stage:
            cp.start()
        for cp in stage:
            cp.wait()
        gs = pltpu.make_async_copy(emb_hbm.at[sidx_v], srows_v, sem)
        gs.start()
        tgt = tgt_v[...]
        cidx_v[pl.ds(0, _G)] = tgt
        col_prev = top_v[0]
        cum = jnp.where(col_prev == tgt, 1, 0)
        for j in range(1, _NC):
            col = top_v[j]
            cidx_v[pl.ds(j * _G, _G)] = jnp.where(cum > 0, col, col_prev)
            cum = cum + jnp.where(col == tgt, 1, 0)
            col_prev = col
        gc = pltpu.make_async_copy(emb_hbm.at[cidx_v], crows_v, sem)
        gc.start()
        gs.wait()
        gc.wait()
        pltpu.sync_copy(crows_v, cand_out.at[grp])
        pltpu.sync_copy(srows_v, seq_out.at[grp])


def _sc_gather(seq_flat, tgt, top_g, item_emb):
    mesh = plsc.VectorSubcoreMesh(core_axis_name="c", subcore_axis_name="s")
    out_type = (
        jax.ShapeDtypeStruct((_NG, _G * _NC, _D), jnp.float32),
        jax.ShapeDtypeStruct((_NG, _G * _L, _D), jnp.float32),
    )
    run = functools.partial(
        pl.kernel, mesh=mesh, out_type=out_type,
        compiler_params=pltpu.CompilerParams(use_tc_tiling_on_sc=False),
        scratch_types=[
            pltpu.VMEM((_NC, _G), jnp.int32),
            pltpu.VMEM((_G,), jnp.int32),
            pltpu.VMEM((_G * _NC,), jnp.int32),
            pltpu.VMEM((_G * _NC, _D), jnp.float32),
            pltpu.VMEM((_G * _L,), jnp.int32),
            pltpu.VMEM((_G * _L, _D), jnp.float32),
            pltpu.SemaphoreType.DMA,
        ],
    )(_sc_body)
    return run(seq_flat, tgt, top_g, item_emb)


def _tc_body(seq_ref, cand_ref, ts_ref, tis_ref, tt_ref, len_ref, noise_ref,
             out_ref, acc_ref):
    gidx = pl.program_id(0)
    ts = ts_ref[...]
    tis = tis_ref[...]
    tt = tt_ref[...]
    eff = jnp.maximum(len_ref[...], 1)
    pos = lax.broadcasted_iota(jnp.int32, (_G, _L), 1)
    mask = (pos < eff).astype(jnp.float32)
    decay = jnp.exp(-jnp.abs(tt - ts) * 0.1)
    w = decay * (1.0 + 0.01 * tis) * mask
    wsum = jnp.sum(w, axis=1, keepdims=True)
    seq = seq_ref[...]
    user = jnp.sum(seq * w[:, :, None], axis=1) / (wsum + 1e-6)
    cand = cand_ref[...]
    logits = jnp.sum(cand * user[None, :, :], axis=2)
    m = jnp.max(logits, axis=0, keepdims=True)
    e = jnp.exp(logits - m)
    actual = e / jnp.sum(e, axis=0, keepdims=True)
    noise = noise_ref[0]
    deno = float(_K) * noise + actual + 1e-6
    kidx = lax.broadcasted_iota(jnp.int32, (_NC, _G), 0)
    likeli = jnp.where(kidx == 0, actual, noise) / deno
    part = jnp.sum(jnp.log(likeli))

    @pl.when(gidx == 0)
    def _init():
        acc_ref[0] = 0.0

    acc_ref[0] += part

    @pl.when(gidx == _NG - 1)
    def _fin():
        out_ref[...] = jnp.full((1, 1), -acc_ref[0] / float(_B * _NC),
                                jnp.float32)


def _tc_score(seq3, cand3, ts, tis, tt2, len2, noise3):
    return pl.pallas_call(
        _tc_body,
        grid=(_NG,),
        in_specs=[
            pl.BlockSpec((_G, _L, _D), lambda g: (g, 0, 0)),
            pl.BlockSpec((_NC, _G, _D), lambda g: (g, 0, 0)),
            pl.BlockSpec((_G, _L), lambda g: (g, 0)),
            pl.BlockSpec((_G, _L), lambda g: (g, 0)),
            pl.BlockSpec((_G, 1), lambda g: (g, 0)),
            pl.BlockSpec((_G, 1), lambda g: (g, 0)),
            pl.BlockSpec((1, _NC, _G), lambda g: (g, 0, 0)),
        ],
        out_specs=pl.BlockSpec((1, 1), lambda g: (0, 0)),
        out_shape=jax.ShapeDtypeStruct((1, 1), jnp.float32),
        scratch_shapes=[pltpu.SMEM((1,), jnp.float32)],
    )(seq3, cand3, ts, tis, tt2, len2, noise3)


def kernel(item_seq, item_seq_len, target_id, time_seq, time_interval_seq,
           target_time, item_emb):
    top_g = jnp.asarray(_TOP_G_NP)
    noise3 = jnp.asarray(_NOISE_G_NP)
    seq_flat = item_seq.astype(jnp.int32).reshape(_B * _L)
    tgt = target_id.astype(jnp.int32)
    cand_emb, seq_emb = _sc_gather(seq_flat, tgt, top_g, item_emb)
    seq3 = seq_emb.reshape(_B, _L, _D)
    cand3 = cand_emb.reshape(_NG * _NC, _G, _D)
    tt2 = target_time.reshape(_B, 1)
    len2 = item_seq_len.astype(jnp.int32).reshape(_B, 1)
    loss = _tc_score(seq3, cand3, time_seq, time_interval_seq, tt2, len2,
                     noise3)
    return loss[0, 0]


# speedup vs baseline: 99.7836x; 1.0921x over previous
import functools

import jax
import jax.numpy as jnp
import numpy as np
from jax import lax
from jax.experimental import pallas as pl
from jax.experimental.pallas import tpu as pltpu
from jax.experimental.pallas import tpu_sc as plsc

_K = 100
_NC = _K + 1
_NI = 100000
_D = 32
_B = 1024
_L = 50
_G = 16
_NG = _B // _G
_GW = 2
_CCH = 13
_SCH = 7


def _threefry_bits_np(start, stop, k0, k1):
    ks0 = np.uint32(k0)
    ks1 = np.uint32(k1)
    ks2 = np.uint32(ks0 ^ ks1 ^ np.uint32(0x1BD11BDA))
    ks = (ks0, ks1, ks2)
    rot = ((13, 15, 26, 6), (17, 29, 16, 24))
    x1 = np.arange(start, stop, dtype=np.uint32)
    x0 = np.full(stop - start, ks0, np.uint32)
    x1 += ks1
    for i in range(5):
        for r in rot[i % 2]:
            x0 += x1
            x1 = (x1 << np.uint32(r)) | (x1 >> np.uint32(32 - r))
            x1 ^= x0
        x0 += ks[(i + 1) % 3]
        x1 += ks[(i + 2) % 3]
        x1 += np.uint32(i + 1)
    return x0 ^ x1


def _bits_to_unit_float_np(bits):
    fb = (bits >> np.uint32(9)) | np.uint32(0x3F800000)
    return fb.view(np.float32) - np.float32(1.0)


def _build_consts():
    top_idx = np.empty((_B, _NC), np.int32)
    tiny = np.float32(np.finfo(np.float32).tiny)
    span = np.float32(1.0) - tiny
    rows_per = 64
    for r0 in range(0, _B, rows_per):
        bits = _threefry_bits_np(r0 * _NI, (r0 + rows_per) * _NI, 0, 42)
        u = _bits_to_unit_float_np(bits) * span + tiny
        u = np.maximum(tiny, u)
        g = -np.log(-np.log(u)).reshape(rows_per, _NI)
        part = np.argpartition(-g, _NC - 1, axis=1)[:, :_NC]
        vals = np.take_along_axis(g, part, axis=1)
        for i in range(rows_per):
            order = np.lexsort((part[i], -vals[i]))
            top_idx[r0 + i] = part[i][order]
    nbits = _threefry_bits_np(0, _B * _NC, 0, 7)
    noise = _bits_to_unit_float_np(nbits).reshape(_B, _NC)
    e = np.exp(noise - noise.max(axis=1, keepdims=True))
    noise_sm = (e / e.sum(axis=1, keepdims=True)).astype(np.float32)
    top_g = np.ascontiguousarray(
        top_idx.reshape(_NG, _G, _NC).transpose(0, 2, 1))
    return top_g, noise_sm


_PERM_NP = ((np.arange(_G * _NC) % _G) * _NC
            + np.arange(_G * _NC) // _G).astype(np.int32)

_TOP_G_NP, _NOISE_NP = _build_consts()


def _sc_body(seq_hbm, tgt_hbm, top_hbm, perm_hbm, emb_hbm, cand_out, seq_out,
             top_v, tgt_v, cidx_v, crows_v, sidx_v, srows_v, perm_v, sem):
    wid = lax.axis_index("s") * 2 + lax.axis_index("c")
    pltpu.sync_copy(perm_hbm, perm_v)
    wb = []
    for gi in range(_GW):
        grp = wid * _GW + gi
        base = grp * _G
        stage = [
            pltpu.make_async_copy(tgt_hbm.at[pl.ds(base, _G)], tgt_v, sem),
            pltpu.make_async_copy(top_hbm.at[grp], top_v, sem),
            pltpu.make_async_copy(
                seq_hbm.at[pl.ds(base * _L, _G * _L)], sidx_v, sem),
        ]
        for cp in stage:
            cp.start()
        for cp in stage:
            cp.wait()
        for cp in wb:
            cp.wait()
        wb = []
        gs = pltpu.make_async_copy(emb_hbm.at[sidx_v], srows_v, sem)
        gs.start()
        tgt = tgt_v[...]
        cidx_v[pl.ds(0, _G)] = tgt
        col_prev = top_v[0]
        cum = jnp.where(col_prev == tgt, 1, 0)
        for j in range(1, _NC):
            col = top_v[j]
            cidx_v[pl.ds(j * _G, _G)] = jnp.where(cum > 0, col, col_prev)
            cum = cum + jnp.where(col == tgt, 1, 0)
            col_prev = col
        gc = pltpu.make_async_copy(emb_hbm.at[cidx_v], crows_v, sem)
        gc.start()
        gs.wait()
        gc.wait()
        wb = [
            pltpu.make_async_copy(
                crows_v,
                cand_out.at[pl.ds(base * _NC, _G * _NC)].at[perm_v], sem),
            pltpu.make_async_copy(
                srows_v, seq_out.at[pl.ds(base * _L, _G * _L)], sem),
        ]
        for cp in wb:
            cp.start()
    for cp in wb:
        cp.wait()


def _sc_gather(seq_flat, tgt, top_g, perm, item_emb):
    mesh = plsc.VectorSubcoreMesh(core_axis_name="c", subcore_axis_name="s")
    out_type = (
        jax.ShapeDtypeStruct((_B * _NC, _D), jnp.float32),
        jax.ShapeDtypeStruct((_B * _L, _D), jnp.float32),
    )
    run = functools.partial(
        pl.kernel, mesh=mesh, out_type=out_type,
        compiler_params=pltpu.CompilerParams(use_tc_tiling_on_sc=False),
        scratch_types=[
            pltpu.VMEM((_NC, _G), jnp.int32),
            pltpu.VMEM((_G,), jnp.int32),
            pltpu.VMEM((_G * _NC,), jnp.int32),
            pltpu.VMEM((_G * _NC, _D), jnp.float32),
            pltpu.VMEM((_G * _L,), jnp.int32),
            pltpu.VMEM((_G * _L, _D), jnp.float32),
            pltpu.VMEM((_G * _NC,), jnp.int32),
            pltpu.SemaphoreType.DMA,
        ],
    )(_sc_body)
    return run(seq_flat, tgt, top_g, perm, item_emb)


_TB = 128
_TNB = _B // _TB


def _tc_body(seq_ref, cand_ref, ts_ref, tis_ref, tt_ref, len_ref, noise_ref,
             out_ref, acc_ref):
    gidx = pl.program_id(0)
    ts = ts_ref[...]
    tis = tis_ref[...]
    tt = tt_ref[...]
    eff = jnp.maximum(len_ref[...], 1)
    pos = lax.broadcasted_iota(jnp.int32, (_TB, _L), 1)
    mask = (pos < eff).astype(jnp.float32)
    decay = jnp.exp(-jnp.abs(tt - ts) * 0.1)
    w = decay * (1.0 + 0.01 * tis) * mask
    wsum = jnp.sum(w, axis=1, keepdims=True)
    seq = seq_ref[...]
    user = jnp.sum(seq * w[:, :, None], axis=1) / (wsum + 1e-6)
    cand = cand_ref[...]
    logits = jnp.sum(cand * user[:, None, :], axis=2)
    m = jnp.max(logits, axis=1, keepdims=True)
    e = jnp.exp(logits - m)
    actual = e / jnp.sum(e, axis=1, keepdims=True)
    noise = noise_ref[...]
    deno = float(_K) * noise + actual + 1e-6
    kidx = lax.broadcasted_iota(jnp.int32, (_TB, _NC), 1)
    likeli = jnp.where(kidx == 0, actual, noise) / deno
    part = jnp.sum(jnp.log(likeli))

    @pl.when(gidx == 0)
    def _init():
        acc_ref[0] = 0.0

    acc_ref[0] += part

    @pl.when(gidx == _TNB - 1)
    def _fin():
        out_ref[...] = jnp.full((1, 1), -acc_ref[0] / float(_B * _NC),
                                jnp.float32)


def _tc_score(seq3, cand3, ts, tis, tt2, len2, noise2):
    return pl.pallas_call(
        _tc_body,
        grid=(_TNB,),
        in_specs=[
            pl.BlockSpec((_TB, _L, _D), lambda g: (g, 0, 0)),
            pl.BlockSpec((_TB, _NC, _D), lambda g: (g, 0, 0)),
            pl.BlockSpec((_TB, _L), lambda g: (g, 0)),
            pl.BlockSpec((_TB, _L), lambda g: (g, 0)),
            pl.BlockSpec((_TB, 1), lambda g: (g, 0)),
            pl.BlockSpec((_TB, 1), lambda g: (g, 0)),
            pl.BlockSpec((_TB, _NC), lambda g: (g, 0)),
        ],
        out_specs=pl.BlockSpec((1, 1), lambda g: (0, 0)),
        out_shape=jax.ShapeDtypeStruct((1, 1), jnp.float32),
        scratch_shapes=[pltpu.SMEM((1,), jnp.float32)],
    )(seq3, cand3, ts, tis, tt2, len2, noise2)


def kernel(item_seq, item_seq_len, target_id, time_seq, time_interval_seq,
           target_time, item_emb):
    top_g = jnp.asarray(_TOP_G_NP)
    noise2 = jnp.asarray(_NOISE_NP)
    perm = jnp.asarray(_PERM_NP)
    seq_flat = item_seq.astype(jnp.int32).reshape(_B * _L)
    tgt = target_id.astype(jnp.int32)
    cand_emb, seq_emb = _sc_gather(seq_flat, tgt, top_g, perm, item_emb)
    cand3 = cand_emb.reshape(_B, _NC, _D)
    seq3 = seq_emb.reshape(_B, _L, _D)
    tt2 = target_time.reshape(_B, 1)
    len2 = item_seq_len.astype(jnp.int32).reshape(_B, 1)
    loss = _tc_score(seq3, cand3, time_seq, time_interval_seq, tt2,
                     len2, noise2)
    return loss[0, 0]
